# baseline (speedup 1.0000x reference)
import functools

import jax
import jax.numpy as jnp
from jax import lax
from jax.experimental import pallas as pl
from jax.experimental.pallas import tpu as pltpu
from jax.experimental.pallas import tpu_sc as plsc

N = 10000
D_IN = 256
D_OUT = 128
E = N * 16

NC = 2
NS = 16
NW = NC * NS
CH = 125
BLK_PER_W = 40
B0 = 40
B1 = 40
NBLK = E // CH
ACC_ROWS = N
RPS = 624
BM = 1000


@functools.cache
def _make_segsum(num_passes):
  mesh = plsc.VectorSubcoreMesh(core_axis_name="c", subcore_axis_name="s")
  bmax = max(B0, B1)

  def body(*refs):
    zeros_hbm = refs[0]
    hs = refs[1:1 + num_passes]
    srcs = refs[1 + num_passes:1 + 2 * num_passes]
    dsts = refs[1 + 2 * num_passes:1 + 3 * num_passes]
    outs = refs[1 + 3 * num_passes:1 + 4 * num_passes]
    (src_v, dst_v, rows_a, rows_b, acc,
     sem_a, sem_b, sem_sa, sem_sb) = refs[1 + 4 * num_passes:1 + 4 * num_passes + 9]

    cid = lax.axis_index("c")
    sid = lax.axis_index("s")
    nblk2 = jnp.where(cid == 0, B0 // 2, B1 // 2)

    for p in range(num_passes):
      pltpu.sync_copy(zeros_hbm.at[pl.ds(sid * RPS, RPS)],
                      acc.at[pl.ds(sid * RPS, RPS)])

      @pl.when(sid == NS - 1)
      def _():
        pltpu.sync_copy(zeros_hbm.at[pl.ds(NS * RPS, 16)],
                        acc.at[pl.ds(NS * RPS, 16)])
      plsc.subcore_barrier()

      @pl.when(cid == 0)
      def _(p=p):
        pltpu.sync_copy(srcs[p].at[pl.ds(sid * B0, B0)],
                        src_v.at[pl.ds(0, B0)])
        pltpu.sync_copy(dsts[p].at[pl.ds(sid * B0, B0)],
                        dst_v.at[pl.ds(0, B0)])

      @pl.when(cid == 1)
      def _(p=p):
        pltpu.sync_copy(srcs[p].at[pl.ds(NS * B0 + sid * B1, B1)],
                        src_v.at[pl.ds(0, B1)])
        pltpu.sync_copy(dsts[p].at[pl.ds(NS * B0 + sid * B1, B1)],
                        dst_v.at[pl.ds(0, B1)])

      h = hs[p]
      pltpu.async_copy(h.at[src_v.at[0]], rows_a, sem_a)

      def edge_body(jj, carry, h=h):
        j0 = jj * 2
        pltpu.async_copy(h.at[src_v.at[j0 + 1]], rows_b, sem_b)
        pltpu.make_async_copy(h.at[src_v.at[j0]], rows_a, sem_a).wait()
        pltpu.async_copy(rows_a, acc.at[dst_v.at[j0]], sem_sa, add=True)
        pltpu.make_async_copy(h.at[src_v.at[j0 + 1]], rows_b, sem_b).wait()
        pltpu.async_copy(rows_b, acc.at[dst_v.at[j0 + 1]], sem_sb, add=True)
        pltpu.make_async_copy(rows_a, acc.at[dst_v.at[j0]], sem_sa).wait()

        @pl.when(jj < nblk2 - 1)
        def _():
          pltpu.async_copy(h.at[src_v.at[j0 + 2]], rows_a, sem_a)
        pltpu.make_async_copy(rows_b, acc.at[dst_v.at[j0 + 1]], sem_sb).wait()

        @pl.when(jj < nblk2 - 1)
        def _():
          pltpu.async_copy(h.at[src_v.at[j0 + 3]], rows_b, sem_b)
        return carry
      lax.fori_loop(0, nblk2, edge_body, 0)

      plsc.subcore_barrier()
      pltpu.sync_copy(acc.at[pl.ds(sid * RPS, RPS)],
                      outs[p].at[cid, pl.ds(sid * RPS, RPS)])

      @pl.when(sid == NS - 1)
      def _():
        pltpu.sync_copy(acc.at[pl.ds(NS * RPS, 16)],
                        outs[p].at[cid, pl.ds(NS * RPS, 16)])

  out_type = [jax.ShapeDtypeStruct((NC, N, D_OUT), jnp.float32)
              for _ in range(num_passes)]
  scratch = [
      pltpu.VMEM((bmax, CH), jnp.int32),
      pltpu.VMEM((bmax, CH), jnp.int32),
      pltpu.VMEM((CH, D_OUT), jnp.float32),
      pltpu.VMEM((CH, D_OUT), jnp.float32),
      pltpu.VMEM_SHARED((ACC_ROWS, D_OUT), jnp.float32),
      pltpu.SemaphoreType.DMA,
      pltpu.SemaphoreType.DMA,
      pltpu.SemaphoreType.DMA,
      pltpu.SemaphoreType.DMA,
  ]
  return pl.kernel(body, out_type=out_type, mesh=mesh, scratch_types=scratch,
                   name=f"sc_segsum_{num_passes}")


def _pad_adj(src, dst):
  return src.reshape(NBLK, CH), dst.reshape(NBLK, CH)


def _mm1_body(x1, x2, x3, w1, w2, w3, b1, b2, b3, o1, o2, o3):
  for x, w, b, o in ((x1, w1, b1, o1), (x2, w2, b2, o2), (x3, w3, b3, o3)):
    o[...] = jnp.dot(x[...], w[...], preferred_element_type=jnp.float32) + b[...]


def _stage_a(xs, w1s, b1s):
  xspec = pl.BlockSpec((BM, D_IN), lambda i: (i, 0))
  wspec = pl.BlockSpec((D_IN, D_OUT), lambda i: (0, 0))
  bspec = pl.BlockSpec((1, D_OUT), lambda i: (0, 0))
  ospec = pl.BlockSpec((BM, D_OUT), lambda i: (i, 0))
  out = [jax.ShapeDtypeStruct((N, D_OUT), jnp.float32)] * 3
  return pl.pallas_call(
      _mm1_body,
      grid=(N // BM,),
      in_specs=[xspec] * 3 + [wspec] * 3 + [bspec] * 3,
      out_specs=[ospec] * 3,
      out_shape=out,
  )(*xs, *w1s, *[b.reshape(1, D_OUT) for b in b1s])


def _layer2_body(p1, p2, p3, h1, w1, w2, w3, b1, b2, b3,
                 l1o1, l1o2, l1o3, h2o1, h2o2, h2o3):
  for t, (p, w, b, lo, ho) in enumerate(
      ((p1, w1, b1, l1o1, h2o1), (p2, w2, b2, l1o2, h2o2),
       (p3, w3, b3, l1o3, h2o3))):
    agg = p[0] + p[1]
    if t == 0:
      agg = agg + h1[...]
    l1 = jnp.maximum(agg, 0.0)
    lo[...] = l1
    ho[...] = jnp.dot(l1, w[...], preferred_element_type=jnp.float32) + b[...]


def _stage_c(parts, h1_1, w2s, b2s):
  pspec = pl.BlockSpec((NC, BM, D_OUT), lambda i: (0, i, 0))
  hspec = pl.BlockSpec((BM, D_OUT), lambda i: (i, 0))
  wspec = pl.BlockSpec((D_OUT, D_OUT), lambda i: (0, 0))
  bspec = pl.BlockSpec((1, D_OUT), lambda i: (0, 0))
  ospec = pl.BlockSpec((BM, D_OUT), lambda i: (i, 0))
  out = [jax.ShapeDtypeStruct((N, D_OUT), jnp.float32)] * 6
  res = pl.pallas_call(
      _layer2_body,
      grid=(N // BM,),
      in_specs=[pspec] * 3 + [hspec] + [wspec] * 3 + [bspec] * 3,
      out_specs=[ospec] * 6,
      out_shape=out,
  )(*parts, h1_1, *w2s, *[b.reshape(1, D_OUT) for b in b2s])
  return res[0:3], res[3:6]


def _relu_sum_body(q1, q2, q3, o1, o2, o3):
  for q, o in ((q1, o1), (q2, o2), (q3, o3)):
    o[...] = jnp.maximum(q[0] + q[1], 0.0)


def _stage_e(parts):
  pspec = pl.BlockSpec((NC, BM, D_OUT), lambda i: (0, i, 0))
  ospec = pl.BlockSpec((BM, D_OUT), lambda i: (i, 0))
  out = [jax.ShapeDtypeStruct((N, D_OUT), jnp.float32)] * 3
  return pl.pallas_call(
      _relu_sum_body,
      grid=(N // BM,),
      in_specs=[pspec] * 3,
      out_specs=[ospec] * 3,
      out_shape=out,
  )(*parts)


def _norm_body(r1, r2, r3, s1, s2, s3, od, os_):
  for t, (r, s) in enumerate(((r1, s1), (r2, s2), (r3, s3))):
    a = r[0] + r[1]
    b = s[0] + s[1]
    an = a / (jnp.sqrt(jnp.sum(a * a, axis=1, keepdims=True)) + 1e-9)
    bn = b / (jnp.sqrt(jnp.sum(b * b, axis=1, keepdims=True)) + 1e-9)
    od[:, t * D_OUT:(t + 1) * D_OUT] = an
    os_[:, t * D_OUT:(t + 1) * D_OUT] = bn


def _stage_g(r_parts, s_parts):
  pspec = pl.BlockSpec((NC, BM, D_OUT), lambda i: (0, i, 0))
  ospec = pl.BlockSpec((BM, 3 * D_OUT), lambda i: (i, 0))
  out = [jax.ShapeDtypeStruct((N, 3 * D_OUT), jnp.float32)] * 2
  return pl.pallas_call(
      _norm_body,
      grid=(N // BM,),
      in_specs=[pspec] * 6,
      out_specs=[ospec] * 2,
      out_shape=out,
  )(*r_parts, *s_parts)


def kernel(x1, x2, x3, W1_1, b1_1, W2_1, b2_1, W1_2, b1_2, W2_2, b2_2,
           W1_3, b1_3, W2_3, b2_3, adj11_src, adj11_dst, adj01_src, adj01_dst,
           adj22_src, adj22_dst, adj02_src, adj02_dst, adj33_src, adj33_dst,
           adj03_src, adj03_dst, epoch):
  del epoch
  intra = [_pad_adj(adj11_src, adj11_dst), _pad_adj(adj22_src, adj22_dst),
           _pad_adj(adj33_src, adj33_dst)]
  cross = [_pad_adj(adj01_src, adj01_dst), _pad_adj(adj02_src, adj02_dst),
           _pad_adj(adj03_src, adj03_dst)]
  zeros = jnp.zeros((ACC_ROWS, D_OUT), jnp.float32)

  h1 = _stage_a((x1, x2, x3), (W1_1, W1_2, W1_3), (b1_1, b1_2, b1_3))

  seg3 = _make_segsum(3)
  p1 = seg3(zeros, *h1, *[a[0] for a in intra], *[a[1] for a in intra])

  l1, h2 = _stage_c(p1, h1[0], (W2_1, W2_2, W2_3), (b2_1, b2_2, b2_3))

  seg6 = _make_segsum(6)
  p2 = seg6(zeros, *h2, *l1,
            *[a[0] for a in intra], *[a[0] for a in cross],
            *[a[1] for a in intra], *[a[1] for a in cross])

  l2 = _stage_e(p2[0:3])

  r_parts = seg3(zeros, *l2, *[a[0] for a in cross], *[a[1] for a in cross])

  doc, doc_svd = _stage_g(r_parts, p2[3:6])
  return doc, doc_svd


# speedup vs baseline: 1.4078x; 1.4078x over previous
import functools

import jax
import jax.numpy as jnp
from jax import lax
from jax.experimental import pallas as pl
from jax.experimental.pallas import tpu as pltpu
from jax.experimental.pallas import tpu_sc as plsc

N = 10000
D_IN = 256
D_OUT = 128
E = N * 16

NC = 2
NS = 16
NW = NC * NS
CH = 125
BLK_PER_W = 40
B0 = 40
B1 = 40
NBLK = E // CH
ACC_ROWS = N
RPS = 624
BM = 1000


@functools.cache
def _make_segsum(num_passes):
  mesh = plsc.VectorSubcoreMesh(core_axis_name="c", subcore_axis_name="s")
  bmax = max(B0, B1)

  def body(*refs):
    zeros_hbm = refs[0]
    hs = refs[1:1 + num_passes]
    srcs = refs[1 + num_passes:1 + 2 * num_passes]
    dsts = refs[1 + 2 * num_passes:1 + 3 * num_passes]
    outs = refs[1 + 3 * num_passes:1 + 4 * num_passes]
    (src_v, dst_v, rows_a, rows_b, acc,
     sem_a, sem_b) = refs[1 + 4 * num_passes:1 + 4 * num_passes + 7]

    cid = lax.axis_index("c")
    sid = lax.axis_index("s")
    nblk2 = jnp.where(cid == 0, B0 // 2, B1 // 2)

    for p in range(num_passes):
      pltpu.sync_copy(zeros_hbm.at[pl.ds(sid * RPS, RPS)],
                      acc.at[pl.ds(sid * RPS, RPS)])

      @pl.when(sid == NS - 1)
      def _():
        pltpu.sync_copy(zeros_hbm.at[pl.ds(NS * RPS, 16)],
                        acc.at[pl.ds(NS * RPS, 16)])
      plsc.subcore_barrier()

      @pl.when(cid == 0)
      def _(p=p):
        pltpu.sync_copy(srcs[p].at[pl.ds(sid * B0, B0)],
                        src_v.at[pl.ds(0, B0)])
        pltpu.sync_copy(dsts[p].at[pl.ds(sid * B0, B0)],
                        dst_v.at[pl.ds(0, B0)])

      @pl.when(cid == 1)
      def _(p=p):
        pltpu.sync_copy(srcs[p].at[pl.ds(NS * B0 + sid * B1, B1)],
                        src_v.at[pl.ds(0, B1)])
        pltpu.sync_copy(dsts[p].at[pl.ds(NS * B0 + sid * B1, B1)],
                        dst_v.at[pl.ds(0, B1)])

      h = hs[p]
      pltpu.async_copy(h.at[src_v.at[0]], rows_a, sem_a)

      def edge_body(jj, carry, h=h):
        j0 = jj * 2
        pltpu.async_copy(h.at[src_v.at[j0 + 1]], rows_b, sem_b)
        pltpu.make_async_copy(h.at[src_v.at[j0]], rows_a, sem_a).wait()
        pltpu.sync_copy(rows_a, acc.at[dst_v.at[j0]], add=True)

        @pl.when(jj < nblk2 - 1)
        def _():
          pltpu.async_copy(h.at[src_v.at[j0 + 2]], rows_a, sem_a)
        pltpu.make_async_copy(h.at[src_v.at[j0 + 1]], rows_b, sem_b).wait()
        pltpu.sync_copy(rows_b, acc.at[dst_v.at[j0 + 1]], add=True)
        return carry
      lax.fori_loop(0, nblk2, edge_body, 0)

      plsc.subcore_barrier()
      pltpu.sync_copy(acc.at[pl.ds(sid * RPS, RPS)],
                      outs[p].at[cid, pl.ds(sid * RPS, RPS)])

      @pl.when(sid == NS - 1)
      def _():
        pltpu.sync_copy(acc.at[pl.ds(NS * RPS, 16)],
                        outs[p].at[cid, pl.ds(NS * RPS, 16)])

  out_type = [jax.ShapeDtypeStruct((NC, N, D_OUT), jnp.float32)
              for _ in range(num_passes)]
  scratch = [
      pltpu.VMEM((bmax, CH), jnp.int32),
      pltpu.VMEM((bmax, CH), jnp.int32),
      pltpu.VMEM((CH, D_OUT), jnp.float32),
      pltpu.VMEM((CH, D_OUT), jnp.float32),
      pltpu.VMEM_SHARED((ACC_ROWS, D_OUT), jnp.float32),
      pltpu.SemaphoreType.DMA,
      pltpu.SemaphoreType.DMA,
  ]
  return pl.kernel(body, out_type=out_type, mesh=mesh, scratch_types=scratch,
                   name=f"sc_segsum_{num_passes}")


def _pad_adj(src, dst):
  return src.reshape(NBLK, CH), dst.reshape(NBLK, CH)


def _mm1_body(x1, x2, x3, w1, w2, w3, b1, b2, b3, o1, o2, o3):
  for x, w, b, o in ((x1, w1, b1, o1), (x2, w2, b2, o2), (x3, w3, b3, o3)):
    o[...] = jnp.dot(x[...], w[...], preferred_element_type=jnp.float32) + b[...]


def _stage_a(xs, w1s, b1s):
  xspec = pl.BlockSpec((BM, D_IN), lambda i: (i, 0))
  wspec = pl.BlockSpec((D_IN, D_OUT), lambda i: (0, 0))
  bspec = pl.BlockSpec((1, D_OUT), lambda i: (0, 0))
  ospec = pl.BlockSpec((BM, D_OUT), lambda i: (i, 0))
  out = [jax.ShapeDtypeStruct((N, D_OUT), jnp.float32)] * 3
  return pl.pallas_call(
      _mm1_body,
      grid=(N // BM,),
      in_specs=[xspec] * 3 + [wspec] * 3 + [bspec] * 3,
      out_specs=[ospec] * 3,
      out_shape=out,
  )(*xs, *w1s, *[b.reshape(1, D_OUT) for b in b1s])


def _layer2_body(p1, p2, p3, h1, w1, w2, w3, b1, b2, b3,
                 l1o1, l1o2, l1o3, h2o1, h2o2, h2o3):
  for t, (p, w, b, lo, ho) in enumerate(
      ((p1, w1, b1, l1o1, h2o1), (p2, w2, b2, l1o2, h2o2),
       (p3, w3, b3, l1o3, h2o3))):
    agg = p[0] + p[1]
    if t == 0:
      agg = agg + h1[...]
    l1 = jnp.maximum(agg, 0.0)
    lo[...] = l1
    ho[...] = jnp.dot(l1, w[...], preferred_element_type=jnp.float32) + b[...]


def _stage_c(parts, h1_1, w2s, b2s):
  pspec = pl.BlockSpec((NC, BM, D_OUT), lambda i: (0, i, 0))
  hspec = pl.BlockSpec((BM, D_OUT), lambda i: (i, 0))
  wspec = pl.BlockSpec((D_OUT, D_OUT), lambda i: (0, 0))
  bspec = pl.BlockSpec((1, D_OUT), lambda i: (0, 0))
  ospec = pl.BlockSpec((BM, D_OUT), lambda i: (i, 0))
  out = [jax.ShapeDtypeStruct((N, D_OUT), jnp.float32)] * 6
  res = pl.pallas_call(
      _layer2_body,
      grid=(N // BM,),
      in_specs=[pspec] * 3 + [hspec] + [wspec] * 3 + [bspec] * 3,
      out_specs=[ospec] * 6,
      out_shape=out,
  )(*parts, h1_1, *w2s, *[b.reshape(1, D_OUT) for b in b2s])
  return res[0:3], res[3:6]


def _relu_sum_body(q1, q2, q3, o1, o2, o3):
  for q, o in ((q1, o1), (q2, o2), (q3, o3)):
    o[...] = jnp.maximum(q[0] + q[1], 0.0)


def _stage_e(parts):
  pspec = pl.BlockSpec((NC, BM, D_OUT), lambda i: (0, i, 0))
  ospec = pl.BlockSpec((BM, D_OUT), lambda i: (i, 0))
  out = [jax.ShapeDtypeStruct((N, D_OUT), jnp.float32)] * 3
  return pl.pallas_call(
      _relu_sum_body,
      grid=(N // BM,),
      in_specs=[pspec] * 3,
      out_specs=[ospec] * 3,
      out_shape=out,
  )(*parts)


def _norm_body(r1, r2, r3, s1, s2, s3, od, os_):
  for t, (r, s) in enumerate(((r1, s1), (r2, s2), (r3, s3))):
    a = r[0] + r[1]
    b = s[0] + s[1]
    an = a / (jnp.sqrt(jnp.sum(a * a, axis=1, keepdims=True)) + 1e-9)
    bn = b / (jnp.sqrt(jnp.sum(b * b, axis=1, keepdims=True)) + 1e-9)
    od[:, t * D_OUT:(t + 1) * D_OUT] = an
    os_[:, t * D_OUT:(t + 1) * D_OUT] = bn


def _stage_g(r_parts, s_parts):
  pspec = pl.BlockSpec((NC, BM, D_OUT), lambda i: (0, i, 0))
  ospec = pl.BlockSpec((BM, 3 * D_OUT), lambda i: (i, 0))
  out = [jax.ShapeDtypeStruct((N, 3 * D_OUT), jnp.float32)] * 2
  return pl.pallas_call(
      _norm_body,
      grid=(N // BM,),
      in_specs=[pspec] * 6,
      out_specs=[ospec] * 2,
      out_shape=out,
  )(*r_parts, *s_parts)


def kernel(x1, x2, x3, W1_1, b1_1, W2_1, b2_1, W1_2, b1_2, W2_2, b2_2,
           W1_3, b1_3, W2_3, b2_3, adj11_src, adj11_dst, adj01_src, adj01_dst,
           adj22_src, adj22_dst, adj02_src, adj02_dst, adj33_src, adj33_dst,
           adj03_src, adj03_dst, epoch):
  del epoch
  intra = [_pad_adj(adj11_src, adj11_dst), _pad_adj(adj22_src, adj22_dst),
           _pad_adj(adj33_src, adj33_dst)]
  cross = [_pad_adj(adj01_src, adj01_dst), _pad_adj(adj02_src, adj02_dst),
           _pad_adj(adj03_src, adj03_dst)]
  zeros = jnp.zeros((ACC_ROWS, D_OUT), jnp.float32)

  h1 = _stage_a((x1, x2, x3), (W1_1, W1_2, W1_3), (b1_1, b1_2, b1_3))

  seg3 = _make_segsum(3)
  p1 = seg3(zeros, *h1, *[a[0] for a in intra], *[a[1] for a in intra])

  l1, h2 = _stage_c(p1, h1[0], (W2_1, W2_2, W2_3), (b2_1, b2_2, b2_3))

  seg6 = _make_segsum(6)
  p2 = seg6(zeros, *h2, *l1,
            *[a[0] for a in intra], *[a[0] for a in cross],
            *[a[1] for a in intra], *[a[1] for a in cross])

  l2 = _stage_e(p2[0:3])

  r_parts = seg3(zeros, *l2, *[a[0] for a in cross], *[a[1] for a in cross])

  doc, doc_svd = _stage_g(r_parts, p2[3:6])
  return doc, doc_svd
